# baseline (speedup 1.0000x reference)
import functools

import jax
import jax.numpy as jnp
import numpy as np
from jax.experimental import pallas as pl

H = 8
HID = 128
DH = HID // H


def _final_matmul_kernel(flat_ref, w_ref, b_ref, o_ref):
    o_ref[...] = flat_ref[...] @ w_ref[...] + b_ref[...]


def _final_matmul(flat, W, b):
    n = flat.shape[0]
    blk = 1000
    return pl.pallas_call(
        _final_matmul_kernel,
        grid=(n // blk,),
        in_specs=[
            pl.BlockSpec((blk, flat.shape[1]), lambda i: (i, 0)),
            pl.BlockSpec((flat.shape[1], W.shape[1]), lambda i: (0, 0)),
            pl.BlockSpec((1, W.shape[1]), lambda i: (0, 0)),
        ],
        out_specs=pl.BlockSpec((blk, W.shape[1]), lambda i: (i, 0)),
        out_shape=jax.ShapeDtypeStruct((n, W.shape[1]), jnp.float32),
    )(flat, W, b.reshape(1, -1))


def kernel(x_paper, x_author, ei_writes, ei_written_by, params):
    def lin(name, x):
        W, b = params[name]
        return x @ W + b

    h_paper = jax.nn.relu(lin('in_paper', x_paper))
    h_author = jax.nn.relu(lin('in_author', x_author))
    n_p = h_paper.shape[0]

    q = lin('q_paper', h_paper).reshape(n_p, H, DH)
    kd = lin('k_author', h_author).reshape(-1, H, DH)
    vd = lin('v_author', h_author).reshape(-1, H, DH)

    kfull = jnp.einsum('nhd,hdr->nhr', kd, params['a_rel_writes'])
    vfull = jnp.einsum('nhd,hdr->nhr', vd, params['m_rel_writes'])

    src, dst = ei_writes[0], ei_writes[1]
    k_j = kfull[src]
    q_i = q[dst]
    v_j = vfull[src]
    alpha = (q_i * k_j).sum(-1) * params['p_rel_writes'][None, :]
    alpha = alpha / np.sqrt(DH)
    m = jax.ops.segment_max(alpha, dst, num_segments=n_p)
    m = jnp.where(jnp.isfinite(m), m, 0.0)
    e = jnp.exp(alpha - m[dst])
    s = jax.ops.segment_sum(e, dst, num_segments=n_p)
    alpha = e / (s[dst] + 1e-16)
    msg = (v_j * alpha[:, :, None]).reshape(-1, H * DH)
    out_p = jax.ops.segment_sum(msg, dst, num_segments=n_p)
    out_p = out_p + params['rel_enc_writes'][None, :]

    o = jax.nn.gelu(out_p, approximate=False)
    Wa, ba = params['a_paper']
    o = o @ Wa + ba
    flat = jnp.concatenate([h_paper, o], axis=1)
    W, b = params['out']
    return _final_matmul(flat, W, b)


# speedup vs baseline: 2.4913x; 2.4913x over previous
import functools

import jax
import jax.numpy as jnp
import numpy as np
from jax import lax
from jax.experimental import pallas as pl
from jax.experimental.pallas import tpu as pltpu
from jax.experimental.pallas import tpu_sc as plsc

H = 8
HID = 128
DH = HID // H

NC = 2
NS = 16
NW = NC * NS
C = 64


def _proj_paper_body(x_ref, wi_ref, bi_ref, wq_ref, bq_ref, h_ref, q_ref):
    h = jax.nn.relu(x_ref[...] @ wi_ref[...] + bi_ref[...])
    h_ref[...] = h
    q_ref[...] = h @ wq_ref[...] + bq_ref[...]


def _proj_author_body(x_ref, wi_ref, bi_ref, wk_ref, bk_ref, wv_ref, bv_ref,
                      k_ref, v_ref):
    h = jax.nn.relu(x_ref[...] @ wi_ref[...] + bi_ref[...])
    k_ref[...] = h @ wk_ref[...] + bk_ref[...]
    v_ref[...] = h @ wv_ref[...] + bv_ref[...]


def _row_blocks(n):
    for b in (1000, 500, 250, 200, 125, 100, 50, 40, 25, 20, 10, 8, 5, 4, 2, 1):
        if n % b == 0:
            return b
    return 1


def _proj_paper(x, wi, bi, wq, bq):
    n, din = x.shape
    blk = _row_blocks(n)
    return pl.pallas_call(
        _proj_paper_body,
        grid=(n // blk,),
        in_specs=[
            pl.BlockSpec((blk, din), lambda i: (i, 0)),
            pl.BlockSpec((din, HID), lambda i: (0, 0)),
            pl.BlockSpec((1, HID), lambda i: (0, 0)),
            pl.BlockSpec((HID, HID), lambda i: (0, 0)),
            pl.BlockSpec((1, HID), lambda i: (0, 0)),
        ],
        out_specs=[
            pl.BlockSpec((blk, HID), lambda i: (i, 0)),
            pl.BlockSpec((blk, HID), lambda i: (i, 0)),
        ],
        out_shape=[
            jax.ShapeDtypeStruct((n, HID), jnp.float32),
            jax.ShapeDtypeStruct((n, HID), jnp.float32),
        ],
    )(x, wi, bi, wq, bq)


def _proj_author(x, wi, bi, wk, bk, wv, bv):
    n, din = x.shape
    blk = _row_blocks(n)
    return pl.pallas_call(
        _proj_author_body,
        grid=(n // blk,),
        in_specs=[
            pl.BlockSpec((blk, din), lambda i: (i, 0)),
            pl.BlockSpec((din, HID), lambda i: (0, 0)),
            pl.BlockSpec((1, HID), lambda i: (0, 0)),
            pl.BlockSpec((HID, HID), lambda i: (0, 0)),
            pl.BlockSpec((1, HID), lambda i: (0, 0)),
            pl.BlockSpec((HID, HID), lambda i: (0, 0)),
            pl.BlockSpec((1, HID), lambda i: (0, 0)),
        ],
        out_specs=[
            pl.BlockSpec((blk, HID), lambda i: (i, 0)),
            pl.BlockSpec((blk, HID), lambda i: (i, 0)),
        ],
        out_shape=[
            jax.ShapeDtypeStruct((n, HID), jnp.float32),
            jax.ShapeDtypeStruct((n, HID), jnp.float32),
        ],
    )(x, wi, bi, wk, bk, wv, bv)


def _edge_body(npad, ew, nch,
               q_hbm, kf_hbm, vf_hbm, src_hbm, dst_hbm,
               acc_hbm, s_hbm, m_hbm, alpha_hbm,
               srcv, dstv, qrows, krows, vrows, ebuf, abuf, mbuf, mall,
               acc_sh, s_sh, msh, sem):
    cid = lax.axis_index("c")
    sid = lax.axis_index("s")
    wid = cid * NS + sid
    ebase = wid * ew
    rows_per = npad // NS
    lane = lax.iota(jnp.int32, 16)
    zero16 = jnp.zeros((16,), jnp.float32)

    for r in range(C):
        ebuf[r, pl.ds(0, 16)] = zero16
    for r in range(C):
        for j in range(8):
            vrows[r, pl.ds(j * 16, 16)] = zero16
    row0 = sid * rows_per
    done = 0
    while done < rows_per:
        nr = min(C, rows_per - done)
        pltpu.sync_copy(vrows.at[pl.ds(0, nr)], acc_sh.at[pl.ds(row0 + done, nr)])
        pltpu.sync_copy(ebuf.at[pl.ds(0, nr)], s_sh.at[pl.ds(row0 + done, nr)])
        done += nr

    def chunk_a(i, mcarry):
        base = ebase + i * C
        pltpu.sync_copy(src_hbm.at[pl.ds(base, C)], srcv)
        pltpu.sync_copy(dst_hbm.at[pl.ds(base, C)], dstv)
        pltpu.async_copy(q_hbm.at[dstv], qrows, sem).wait()
        pltpu.async_copy(kf_hbm.at[srcv], krows, sem).wait()
        for g in range(C // 16):
            lids = lane + (g * 16)
            for h in range(H):
                def dbody(t, acc, _h=h, _lids=lids):
                    col0 = jnp.broadcast_to(_h * DH + t * 4, (16,)).astype(jnp.int32)
                    for dd in range(4):
                        qv = plsc.load_gather(qrows, [_lids, col0 + dd])
                        kv = plsc.load_gather(krows, [_lids, col0 + dd])
                        acc = acc + qv * kv
                    return acc
                acc = lax.fori_loop(0, DH // 4, dbody, zero16)
                plsc.store_scatter(abuf, [lids * H + h], acc)
        pltpu.sync_copy(abuf, alpha_hbm.at[pl.ds(base * H, C * H)])
        def mbody(r, mm):
            return jnp.maximum(mm, abuf[pl.ds(r * 16, 16)])
        return lax.fori_loop(0, C // 2, mbody, mcarry)

    mfin = lax.fori_loop(0, nch, chunk_a, zero16)

    mvec = zero16
    for h in range(H):
        mh = jnp.maximum(mfin[h], mfin[h + 8])
        mvec = jnp.where(lane == h, mh, mvec)
    mbuf[...] = mvec
    pltpu.sync_copy(mbuf, msh.at[sid])
    plsc.subcore_barrier()
    pltpu.sync_copy(msh, mall)
    msc = mall[0, pl.ds(0, 16)]
    for j in range(1, NS):
        msc = jnp.maximum(msc, mall[j, pl.ds(0, 16)])
    mbuf[...] = msc
    mh_scalar = [msc[h] for h in range(H)]

    def chunk_b(i, carry):
        base = ebase + i * C
        pltpu.sync_copy(src_hbm.at[pl.ds(base, C)], srcv)
        pltpu.sync_copy(dst_hbm.at[pl.ds(base, C)], dstv)
        pltpu.async_copy(vf_hbm.at[srcv], vrows, sem).wait()
        pltpu.sync_copy(alpha_hbm.at[pl.ds(base * H, C * H)], abuf)
        for g in range(C // 16):
            lids = lane + (g * 16)
            for h in range(H):
                hcol = jnp.full((16,), h, jnp.int32)
                av = plsc.load_gather(abuf, [lids * H + h])
                ev = jnp.exp(av - mh_scalar[h])
                plsc.store_scatter(ebuf, [lids, hcol], ev)
                def dbody(t, carry, _h=h, _lids=lids, _ev=ev):
                    col0 = jnp.broadcast_to(_h * DH + t * 4, (16,)).astype(jnp.int32)
                    for dd in range(4):
                        mv = plsc.load_gather(vrows, [_lids, col0 + dd])
                        plsc.store_scatter(vrows, [_lids, col0 + dd], mv * _ev)
                    return carry
                lax.fori_loop(0, DH // 4, dbody, 0)
        pltpu.sync_copy(vrows, acc_sh.at[dstv], add=True)
        pltpu.sync_copy(ebuf, s_sh.at[dstv], add=True)
        return carry

    lax.fori_loop(0, nch, chunk_b, 0)
    plsc.subcore_barrier()

    pltpu.sync_copy(acc_sh.at[pl.ds(row0, rows_per)],
                    acc_hbm.at[cid, pl.ds(row0, rows_per)])
    pltpu.sync_copy(s_sh.at[pl.ds(row0, rows_per)],
                    s_hbm.at[cid, pl.ds(row0, rows_per)])

    @pl.when(sid == 0)
    def _():
        pltpu.sync_copy(mbuf, m_hbm.at[cid])


def _edge_phase(q_pad, kf, vf, srcp, dstp, npad, ew, nch):
    mesh = plsc.VectorSubcoreMesh(core_axis_name="c", subcore_axis_name="s")
    body = functools.partial(_edge_body, npad, ew, nch)
    f = pl.kernel(
        body,
        compiler_params=pltpu.CompilerParams(
            needs_layout_passes=False, use_tc_tiling_on_sc=False),
        out_type=[
            jax.ShapeDtypeStruct((NC, npad, HID), jnp.float32),
            jax.ShapeDtypeStruct((NC, npad, 16), jnp.float32),
            jax.ShapeDtypeStruct((NC, 16), jnp.float32),
            jax.ShapeDtypeStruct((ew * NW * H,), jnp.float32),
        ],
        mesh=mesh,
        scratch_types=[
            pltpu.VMEM((C,), jnp.int32),
            pltpu.VMEM((C,), jnp.int32),
            pltpu.VMEM((C, HID), jnp.float32),
            pltpu.VMEM((C, HID), jnp.float32),
            pltpu.VMEM((C, HID), jnp.float32),
            pltpu.VMEM((C, 16), jnp.float32),
            pltpu.VMEM((C * H,), jnp.float32),
            pltpu.VMEM((16,), jnp.float32),
            pltpu.VMEM((NS, 16), jnp.float32),
            pltpu.VMEM_SHARED((npad, HID), jnp.float32),
            pltpu.VMEM_SHARED((npad, 16), jnp.float32),
            pltpu.VMEM_SHARED((NS, 16), jnp.float32),
            pltpu.SemaphoreType.DMA,
        ],
    )
    return f(q_pad, kf, vf, srcp, dstp)


def _finish_body(acc_ref, s_ref, hp_ref, sc0_ref, sc1_ref, ss0_ref, ss1_ref,
                 r_ref, rel_ref, wa_ref, ba_ref, w1_ref, w2_ref, bo_ref,
                 o_ref):
    a = acc_ref[0] * sc0_ref[...] + acc_ref[1] * sc1_ref[...]
    s = s_ref[0] * ss0_ref[...] + s_ref[1] * ss1_ref[...]
    den = s @ r_ref[...] + 1e-16
    outp = a / den + rel_ref[...]
    o = 0.5 * outp * (1.0 + lax.erf(outp / np.sqrt(2.0).astype(np.float32)))
    o2 = o @ wa_ref[...] + ba_ref[...]
    o_ref[...] = hp_ref[...] @ w1_ref[...] + o2 @ w2_ref[...] + bo_ref[...]


def _finish(acc, s, hp, sc, ss, rmat, rel, wa, ba, w1, w2, bo):
    n = hp.shape[0]
    dout = w1.shape[1]
    blk = _row_blocks(n)
    return pl.pallas_call(
        _finish_body,
        grid=(n // blk,),
        in_specs=[
            pl.BlockSpec((NC, blk, HID), lambda i: (0, i, 0)),
            pl.BlockSpec((NC, blk, 16), lambda i: (0, i, 0)),
            pl.BlockSpec((blk, HID), lambda i: (i, 0)),
            pl.BlockSpec((1, HID), lambda i: (0, 0)),
            pl.BlockSpec((1, HID), lambda i: (0, 0)),
            pl.BlockSpec((1, 16), lambda i: (0, 0)),
            pl.BlockSpec((1, 16), lambda i: (0, 0)),
            pl.BlockSpec((16, HID), lambda i: (0, 0)),
            pl.BlockSpec((1, HID), lambda i: (0, 0)),
            pl.BlockSpec((HID, HID), lambda i: (0, 0)),
            pl.BlockSpec((1, HID), lambda i: (0, 0)),
            pl.BlockSpec((HID, dout), lambda i: (0, 0)),
            pl.BlockSpec((HID, dout), lambda i: (0, 0)),
            pl.BlockSpec((1, dout), lambda i: (0, 0)),
        ],
        out_specs=pl.BlockSpec((blk, dout), lambda i: (i, 0)),
        out_shape=jax.ShapeDtypeStruct((n, dout), jnp.float32),
    )(acc, s, hp, sc[0:1], sc[1:2], ss[0:1], ss[1:2], rmat, rel, wa, ba,
      w1, w2, bo)


def _block_diag(a):
    bd = jnp.zeros((H, DH, H, DH), jnp.float32)
    bd = bd.at[jnp.arange(H), :, jnp.arange(H), :].set(a)
    return bd.reshape(HID, HID)


def kernel(x_paper, x_author, ei_writes, ei_written_by, params):
    n_p = x_paper.shape[0]
    e = ei_writes.shape[1]

    wi_p, bi_p = params['in_paper']
    wi_a, bi_a = params['in_author']
    wq, bq = params['q_paper']
    wk, bk = params['k_author']
    wv, bv = params['v_author']
    bda = _block_diag(params['a_rel_writes'])
    bdm = _block_diag(params['m_rel_writes'])
    qscale = jnp.repeat(params['p_rel_writes'], DH) / np.sqrt(DH).astype(np.float32)
    wq_f = wq * qscale[None, :]
    bq_f = (bq * qscale)[None, :]
    wk_f = wk @ bda
    bk_f = (bk @ bda)[None, :]
    wv_f = wv @ bdm
    bv_f = (bv @ bdm)[None, :]

    h_p, q = _proj_paper(x_paper, wi_p, bi_p[None, :], wq_f, bq_f)
    kf, vf = _proj_author(x_author, wi_a, bi_a[None, :], wk_f, bk_f, wv_f, bv_f)

    ew = -(-e // (NW * C)) * C
    e_pad = ew * NW
    npad = -(-(n_p + 1) // 128) * 128
    src = ei_writes[0]
    dst = ei_writes[1]
    if e_pad > e:
        src = jnp.concatenate([src, jnp.zeros((e_pad - e,), src.dtype)])
        dst = jnp.concatenate([dst, jnp.full((e_pad - e,), n_p, dst.dtype)])
    q_pad = jnp.concatenate([q, jnp.zeros((npad - n_p, HID), jnp.float32)])

    acc, s, m, _ = _edge_phase(q_pad, kf, vf, src, dst, npad, ew, ew // C)

    mmax = jnp.max(m, axis=0)
    ss = jnp.exp(m - mmax[None, :])
    sc = jnp.repeat(ss[:, :H], DH, axis=1)
    rmat = jnp.repeat(jnp.eye(16, dtype=jnp.float32)[:, :H], DH, axis=1)

    wa, ba = params['a_paper']
    wo, bo = params['out']
    out = _finish(acc, s, h_p,
                  sc, ss, rmat,
                  params['rel_enc_writes'][None, :],
                  wa, ba[None, :], wo[:HID], wo[HID:], bo[None, :])
    return out


# speedup vs baseline: 3.4411x; 1.3813x over previous
import functools

import jax
import jax.numpy as jnp
import numpy as np
from jax import lax
from jax.experimental import pallas as pl
from jax.experimental.pallas import tpu as pltpu
from jax.experimental.pallas import tpu_sc as plsc

H = 8
HID = 128
DH = HID // H

NC = 2
NS = 16
NW = NC * NS
C = 64


def _proj_paper_body(x_ref, wi_ref, bi_ref, wq_ref, bq_ref, h_ref, q_ref):
    h = jax.nn.relu(x_ref[...] @ wi_ref[...] + bi_ref[...])
    h_ref[...] = h
    q_ref[...] = (h @ wq_ref[...] + bq_ref[...]).astype(jnp.bfloat16)


def _proj_author_body(x_ref, wi_ref, bi_ref, wk_ref, bk_ref, wv_ref, bv_ref,
                      k_ref, v_ref):
    h = jax.nn.relu(x_ref[...] @ wi_ref[...] + bi_ref[...])
    k_ref[...] = (h @ wk_ref[...] + bk_ref[...]).astype(jnp.bfloat16)
    v_ref[...] = (h @ wv_ref[...] + bv_ref[...]).astype(jnp.bfloat16)


def _row_blocks(n):
    for b in (1000, 500, 250, 200, 125, 100, 50, 40, 25, 20, 10, 8, 5, 4, 2, 1):
        if n % b == 0:
            return b
    return 1


def _proj_paper(x, wi, bi, wq, bq):
    n, din = x.shape
    blk = _row_blocks(n)
    return pl.pallas_call(
        _proj_paper_body,
        grid=(n // blk,),
        in_specs=[
            pl.BlockSpec((blk, din), lambda i: (i, 0)),
            pl.BlockSpec((din, HID), lambda i: (0, 0)),
            pl.BlockSpec((1, HID), lambda i: (0, 0)),
            pl.BlockSpec((HID, HID), lambda i: (0, 0)),
            pl.BlockSpec((1, HID), lambda i: (0, 0)),
        ],
        out_specs=[
            pl.BlockSpec((blk, HID), lambda i: (i, 0)),
            pl.BlockSpec((blk, HID), lambda i: (i, 0)),
        ],
        out_shape=[
            jax.ShapeDtypeStruct((n, HID), jnp.float32),
            jax.ShapeDtypeStruct((n, HID), jnp.bfloat16),
        ],
    )(x, wi, bi, wq, bq)


def _proj_author(x, wi, bi, wk, bk, wv, bv):
    n, din = x.shape
    blk = _row_blocks(n)
    return pl.pallas_call(
        _proj_author_body,
        grid=(n // blk,),
        in_specs=[
            pl.BlockSpec((blk, din), lambda i: (i, 0)),
            pl.BlockSpec((din, HID), lambda i: (0, 0)),
            pl.BlockSpec((1, HID), lambda i: (0, 0)),
            pl.BlockSpec((HID, HID), lambda i: (0, 0)),
            pl.BlockSpec((1, HID), lambda i: (0, 0)),
            pl.BlockSpec((HID, HID), lambda i: (0, 0)),
            pl.BlockSpec((1, HID), lambda i: (0, 0)),
        ],
        out_specs=[
            pl.BlockSpec((blk, HID), lambda i: (i, 0)),
            pl.BlockSpec((blk, HID), lambda i: (i, 0)),
        ],
        out_shape=[
            jax.ShapeDtypeStruct((n, HID), jnp.bfloat16),
            jax.ShapeDtypeStruct((n, HID), jnp.bfloat16),
        ],
    )(x, wi, bi, wk, bk, wv, bv)


def _edge_body(npad, ew, nch,
               q_hbm, kf_hbm, vf_hbm, src_hbm, dst_hbm,
               acc_hbm, s_hbm, m_hbm, alpha_hbm,
               srcv, dstv, qrows, krows, vrows, msgb, ebuf, abuf, mbuf, mall,
               acc_sh, s_sh, msh, sem):
    cid = lax.axis_index("c")
    sid = lax.axis_index("s")
    wid = cid * NS + sid
    ebase = wid * ew
    rows_per = npad // NS
    lane = lax.iota(jnp.int32, 16)
    zero16 = jnp.zeros((16,), jnp.float32)

    for r in range(C):
        ebuf[r, pl.ds(0, 16)] = zero16
    for r in range(C):
        for j in range(8):
            msgb[r, pl.ds(j * 16, 16)] = zero16
    row0 = sid * rows_per
    done = 0
    while done < rows_per:
        nr = min(C, rows_per - done)
        pltpu.sync_copy(msgb.at[pl.ds(0, nr)], acc_sh.at[pl.ds(row0 + done, nr)])
        pltpu.sync_copy(ebuf.at[pl.ds(0, nr)], s_sh.at[pl.ds(row0 + done, nr)])
        done += nr

    def chunk_a(i, mcarry):
        base = ebase + i * C
        pltpu.sync_copy(src_hbm.at[pl.ds(base, C)], srcv)
        pltpu.sync_copy(dst_hbm.at[pl.ds(base, C)], dstv)
        pltpu.async_copy(q_hbm.at[dstv], qrows, sem).wait()
        pltpu.async_copy(kf_hbm.at[srcv], krows, sem).wait()
        for g in range(C // 16):
            lids = lane + (g * 16)
            for h in range(H):
                def dbody(t, acc, _h=h, _lids=lids):
                    col0 = jnp.broadcast_to(_h * (DH // 2) + t * 4,
                                            (16,)).astype(jnp.int32)
                    for dd in range(4):
                        qw = plsc.load_gather(qrows, [_lids, col0 + dd])
                        kw = plsc.load_gather(krows, [_lids, col0 + dd])
                        qa, qb = plsc.unpack(plsc.bitcast(qw, jnp.bfloat16),
                                             format=plsc.PackFormat.INTERLEAVED)
                        ka, kb = plsc.unpack(plsc.bitcast(kw, jnp.bfloat16),
                                             format=plsc.PackFormat.INTERLEAVED)
                        acc = acc + qa * ka + qb * kb
                    return acc
                acc = lax.fori_loop(0, DH // 8, dbody, zero16)
                plsc.store_scatter(abuf, [lids * H + h], acc)
        pltpu.sync_copy(abuf, alpha_hbm.at[pl.ds(base * H, C * H)])
        def mbody(r, mm):
            return jnp.maximum(mm, abuf[pl.ds(r * 16, 16)])
        return lax.fori_loop(0, C // 2, mbody, mcarry)

    mfin = lax.fori_loop(0, nch, chunk_a, zero16)

    mvec = zero16
    for h in range(H):
        mh = jnp.maximum(mfin[h], mfin[h + 8])
        mvec = jnp.where(lane == h, mh, mvec)
    mbuf[...] = mvec
    pltpu.sync_copy(mbuf, msh.at[sid])
    plsc.subcore_barrier()
    pltpu.sync_copy(msh, mall)
    msc = mall[0, pl.ds(0, 16)]
    for j in range(1, NS):
        msc = jnp.maximum(msc, mall[j, pl.ds(0, 16)])
    mbuf[...] = msc
    mh_scalar = [msc[h] for h in range(H)]

    def chunk_b(i, carry):
        base = ebase + i * C
        pltpu.sync_copy(src_hbm.at[pl.ds(base, C)], srcv)
        pltpu.sync_copy(dst_hbm.at[pl.ds(base, C)], dstv)
        pltpu.async_copy(vf_hbm.at[srcv], vrows, sem).wait()
        pltpu.sync_copy(alpha_hbm.at[pl.ds(base * H, C * H)], abuf)
        for g in range(C // 16):
            lids = lane + (g * 16)
            for h in range(H):
                hcol = jnp.full((16,), h, jnp.int32)
                av = plsc.load_gather(abuf, [lids * H + h])
                ev = jnp.exp(av - mh_scalar[h])
                plsc.store_scatter(ebuf, [lids, hcol], ev)
                def dbody(t, carry, _h=h, _lids=lids, _ev=ev):
                    colp = jnp.broadcast_to(_h * (DH // 2) + t * 4,
                                            (16,)).astype(jnp.int32)
                    colm = jnp.broadcast_to(_h * DH + t * 8,
                                            (16,)).astype(jnp.int32)
                    for dd in range(4):
                        vw = plsc.load_gather(vrows, [_lids, colp + dd])
                        va, vb = plsc.unpack(plsc.bitcast(vw, jnp.bfloat16),
                                             format=plsc.PackFormat.INTERLEAVED)
                        plsc.store_scatter(msgb, [_lids, colm + 2 * dd], va * _ev)
                        plsc.store_scatter(msgb, [_lids, colm + 2 * dd + 1], vb * _ev)
                    return carry
                lax.fori_loop(0, DH // 8, dbody, 0)
        pltpu.sync_copy(msgb, acc_sh.at[dstv], add=True)
        pltpu.sync_copy(ebuf, s_sh.at[dstv], add=True)
        return carry

    lax.fori_loop(0, nch, chunk_b, 0)
    plsc.subcore_barrier()

    pltpu.sync_copy(acc_sh.at[pl.ds(row0, rows_per)],
                    acc_hbm.at[cid, pl.ds(row0, rows_per)])
    pltpu.sync_copy(s_sh.at[pl.ds(row0, rows_per)],
                    s_hbm.at[cid, pl.ds(row0, rows_per)])

    @pl.when(sid == 0)
    def _():
        pltpu.sync_copy(mbuf, m_hbm.at[cid])


def _edge_phase(q_pad, kf, vf, srcp, dstp, npad, ew, nch):
    mesh = plsc.VectorSubcoreMesh(core_axis_name="c", subcore_axis_name="s")
    body = functools.partial(_edge_body, npad, ew, nch)
    f = pl.kernel(
        body,
        compiler_params=pltpu.CompilerParams(
            needs_layout_passes=False, use_tc_tiling_on_sc=False),
        out_type=[
            jax.ShapeDtypeStruct((NC, npad, HID), jnp.float32),
            jax.ShapeDtypeStruct((NC, npad, 16), jnp.float32),
            jax.ShapeDtypeStruct((NC, 16), jnp.float32),
            jax.ShapeDtypeStruct((ew * NW * H,), jnp.float32),
        ],
        mesh=mesh,
        scratch_types=[
            pltpu.VMEM((C,), jnp.int32),
            pltpu.VMEM((C,), jnp.int32),
            pltpu.VMEM((C, HID // 2), jnp.int32),
            pltpu.VMEM((C, HID // 2), jnp.int32),
            pltpu.VMEM((C, HID // 2), jnp.int32),
            pltpu.VMEM((C, HID), jnp.float32),
            pltpu.VMEM((C, 16), jnp.float32),
            pltpu.VMEM((C * H,), jnp.float32),
            pltpu.VMEM((16,), jnp.float32),
            pltpu.VMEM((NS, 16), jnp.float32),
            pltpu.VMEM_SHARED((npad, HID), jnp.float32),
            pltpu.VMEM_SHARED((npad, 16), jnp.float32),
            pltpu.VMEM_SHARED((NS, 16), jnp.float32),
            pltpu.SemaphoreType.DMA,
        ],
    )
    return f(q_pad, kf, vf, srcp, dstp)


def _finish_body(acc_ref, s_ref, hp_ref, sc0_ref, sc1_ref, ss0_ref, ss1_ref,
                 r_ref, rel_ref, wa_ref, ba_ref, w1_ref, w2_ref, bo_ref,
                 o_ref):
    a = acc_ref[0] * sc0_ref[...] + acc_ref[1] * sc1_ref[...]
    s = s_ref[0] * ss0_ref[...] + s_ref[1] * ss1_ref[...]
    den = s @ r_ref[...] + 1e-16
    outp = a / den + rel_ref[...]
    o = 0.5 * outp * (1.0 + lax.erf(outp / np.sqrt(2.0).astype(np.float32)))
    o2 = o @ wa_ref[...] + ba_ref[...]
    o_ref[...] = hp_ref[...] @ w1_ref[...] + o2 @ w2_ref[...] + bo_ref[...]


def _finish(acc, s, hp, sc, ss, rmat, rel, wa, ba, w1, w2, bo):
    n = hp.shape[0]
    dout = w1.shape[1]
    blk = _row_blocks(n)
    return pl.pallas_call(
        _finish_body,
        grid=(n // blk,),
        in_specs=[
            pl.BlockSpec((NC, blk, HID), lambda i: (0, i, 0)),
            pl.BlockSpec((NC, blk, 16), lambda i: (0, i, 0)),
            pl.BlockSpec((blk, HID), lambda i: (i, 0)),
            pl.BlockSpec((1, HID), lambda i: (0, 0)),
            pl.BlockSpec((1, HID), lambda i: (0, 0)),
            pl.BlockSpec((1, 16), lambda i: (0, 0)),
            pl.BlockSpec((1, 16), lambda i: (0, 0)),
            pl.BlockSpec((16, HID), lambda i: (0, 0)),
            pl.BlockSpec((1, HID), lambda i: (0, 0)),
            pl.BlockSpec((HID, HID), lambda i: (0, 0)),
            pl.BlockSpec((1, HID), lambda i: (0, 0)),
            pl.BlockSpec((HID, dout), lambda i: (0, 0)),
            pl.BlockSpec((HID, dout), lambda i: (0, 0)),
            pl.BlockSpec((1, dout), lambda i: (0, 0)),
        ],
        out_specs=pl.BlockSpec((blk, dout), lambda i: (i, 0)),
        out_shape=jax.ShapeDtypeStruct((n, dout), jnp.float32),
    )(acc, s, hp, sc[0:1], sc[1:2], ss[0:1], ss[1:2], rmat, rel, wa, ba,
      w1, w2, bo)


def _block_diag(a):
    bd = jnp.zeros((H, DH, H, DH), jnp.float32)
    bd = bd.at[jnp.arange(H), :, jnp.arange(H), :].set(a)
    return bd.reshape(HID, HID)


def kernel(x_paper, x_author, ei_writes, ei_written_by, params):
    n_p = x_paper.shape[0]
    e = ei_writes.shape[1]

    wi_p, bi_p = params['in_paper']
    wi_a, bi_a = params['in_author']
    wq, bq = params['q_paper']
    wk, bk = params['k_author']
    wv, bv = params['v_author']
    bda = _block_diag(params['a_rel_writes'])
    bdm = _block_diag(params['m_rel_writes'])
    qscale = jnp.repeat(params['p_rel_writes'], DH) / np.sqrt(DH).astype(np.float32)
    wq_f = wq * qscale[None, :]
    bq_f = (bq * qscale)[None, :]
    wk_f = wk @ bda
    bk_f = (bk @ bda)[None, :]
    wv_f = wv @ bdm
    bv_f = (bv @ bdm)[None, :]

    h_p, q = _proj_paper(x_paper, wi_p, bi_p[None, :], wq_f, bq_f)
    kf, vf = _proj_author(x_author, wi_a, bi_a[None, :], wk_f, bk_f, wv_f, bv_f)

    ew = -(-e // (NW * C)) * C
    e_pad = ew * NW
    npad = -(-(n_p + 1) // 128) * 128
    src = ei_writes[0]
    dst = ei_writes[1]
    if e_pad > e:
        src = jnp.concatenate([src, jnp.zeros((e_pad - e,), src.dtype)])
        dst = jnp.concatenate([dst, jnp.full((e_pad - e,), n_p, dst.dtype)])
    q_pad = jnp.concatenate([q, jnp.zeros((npad - n_p, HID), jnp.bfloat16)])
    q_i32 = lax.bitcast_convert_type(
        q_pad.reshape(npad, HID // 2, 2), jnp.int32)
    kf_i32 = lax.bitcast_convert_type(
        kf.reshape(-1, HID // 2, 2), jnp.int32)
    vf_i32 = lax.bitcast_convert_type(
        vf.reshape(-1, HID // 2, 2), jnp.int32)

    acc, s, m, _ = _edge_phase(q_i32, kf_i32, vf_i32, src, dst, npad, ew,
                               ew // C)

    mmax = jnp.max(m, axis=0)
    ss = jnp.exp(m - mmax[None, :])
    sc = jnp.repeat(ss[:, :H], DH, axis=1)
    rmat = jnp.repeat(jnp.eye(16, dtype=jnp.float32)[:, :H], DH, axis=1)

    wa, ba = params['a_paper']
    wo, bo = params['out']
    out = _finish(acc, s, h_p,
                  sc, ss, rmat,
                  params['rel_enc_writes'][None, :],
                  wa, ba[None, :], wo[:HID], wo[HID:], bo[None, :])
    return out


# speedup vs baseline: 3.7223x; 1.0817x over previous
import functools

import jax
import jax.numpy as jnp
import numpy as np
from jax import lax
from jax.experimental import pallas as pl
from jax.experimental.pallas import tpu as pltpu
from jax.experimental.pallas import tpu_sc as plsc

H = 8
HID = 128
DH = HID // H

NC = 2
NS = 16
NW = NC * NS
C = 64


def _proj_paper_body(x_ref, wi_ref, bi_ref, wq_ref, bq_ref, h_ref, q_ref):
    h = jax.nn.relu(x_ref[...] @ wi_ref[...] + bi_ref[...])
    h_ref[...] = h
    q_ref[...] = (h @ wq_ref[...] + bq_ref[...]).astype(jnp.bfloat16)


def _proj_author_body(x_ref, wi_ref, bi_ref, wk_ref, bk_ref, wv_ref, bv_ref,
                      k_ref, v_ref):
    h = jax.nn.relu(x_ref[...] @ wi_ref[...] + bi_ref[...])
    k_ref[...] = (h @ wk_ref[...] + bk_ref[...]).astype(jnp.bfloat16)
    v_ref[...] = (h @ wv_ref[...] + bv_ref[...]).astype(jnp.bfloat16)


def _row_blocks(n):
    for b in (1000, 500, 250, 200, 125, 100, 50, 40, 25, 20, 10, 8, 5, 4, 2, 1):
        if n % b == 0:
            return b
    return 1


def _proj_paper(x, wi, bi, wq, bq):
    n, din = x.shape
    blk = _row_blocks(n)
    return pl.pallas_call(
        _proj_paper_body,
        grid=(n // blk,),
        in_specs=[
            pl.BlockSpec((blk, din), lambda i: (i, 0)),
            pl.BlockSpec((din, HID), lambda i: (0, 0)),
            pl.BlockSpec((1, HID), lambda i: (0, 0)),
            pl.BlockSpec((HID, HID), lambda i: (0, 0)),
            pl.BlockSpec((1, HID), lambda i: (0, 0)),
        ],
        out_specs=[
            pl.BlockSpec((blk, HID), lambda i: (i, 0)),
            pl.BlockSpec((blk, HID), lambda i: (i, 0)),
        ],
        out_shape=[
            jax.ShapeDtypeStruct((n, HID), jnp.float32),
            jax.ShapeDtypeStruct((n, HID), jnp.bfloat16),
        ],
    )(x, wi, bi, wq, bq)


def _proj_author(x, wi, bi, wk, bk, wv, bv):
    n, din = x.shape
    blk = _row_blocks(n)
    return pl.pallas_call(
        _proj_author_body,
        grid=(n // blk,),
        in_specs=[
            pl.BlockSpec((blk, din), lambda i: (i, 0)),
            pl.BlockSpec((din, HID), lambda i: (0, 0)),
            pl.BlockSpec((1, HID), lambda i: (0, 0)),
            pl.BlockSpec((HID, HID), lambda i: (0, 0)),
            pl.BlockSpec((1, HID), lambda i: (0, 0)),
            pl.BlockSpec((HID, HID), lambda i: (0, 0)),
            pl.BlockSpec((1, HID), lambda i: (0, 0)),
        ],
        out_specs=[
            pl.BlockSpec((blk, HID), lambda i: (i, 0)),
            pl.BlockSpec((blk, HID), lambda i: (i, 0)),
        ],
        out_shape=[
            jax.ShapeDtypeStruct((n, HID), jnp.bfloat16),
            jax.ShapeDtypeStruct((n, HID), jnp.bfloat16),
        ],
    )(x, wi, bi, wk, bk, wv, bv)


def _edge_body(npad, ew, nch,
               q_hbm, kf_hbm, vf_hbm, src_hbm, dst_hbm,
               acc_hbm, s_hbm, m_hbm,
               srcv, dstv, qrows, krows, vrows, msgb, ebuf, abuf, mbuf, mall,
               acc_sh, s_sh, msh, sem):
    cid = lax.axis_index("c")
    sid = lax.axis_index("s")
    wid = cid * NS + sid
    ebase = wid * ew
    rows_per = npad // NS
    lane = lax.iota(jnp.int32, 16)
    zero16 = jnp.zeros((16,), jnp.float32)

    for r in range(C):
        ebuf[r, pl.ds(0, 16)] = zero16
    for r in range(C):
        for j in range(8):
            msgb[r, pl.ds(j * 16, 16)] = zero16
    row0 = sid * rows_per
    done = 0
    while done < rows_per:
        nr = min(C, rows_per - done)
        pltpu.sync_copy(msgb.at[pl.ds(0, nr)], acc_sh.at[pl.ds(row0 + done, nr)])
        pltpu.sync_copy(ebuf.at[pl.ds(0, nr)], s_sh.at[pl.ds(row0 + done, nr)])
        done += nr

    def alpha_chunk(compute):
        for g in range(C // 16):
            lids = lane + (g * 16)
            for h in range(H):
                def dbody(t, acc, _h=h, _lids=lids):
                    col0 = jnp.broadcast_to(_h * (DH // 2) + t * 4,
                                            (16,)).astype(jnp.int32)
                    for dd in range(4):
                        qw = plsc.load_gather(qrows, [_lids, col0 + dd])
                        kw = plsc.load_gather(krows, [_lids, col0 + dd])
                        qa, qb = plsc.unpack(plsc.bitcast(qw, jnp.bfloat16),
                                             format=plsc.PackFormat.INTERLEAVED)
                        ka, kb = plsc.unpack(plsc.bitcast(kw, jnp.bfloat16),
                                             format=plsc.PackFormat.INTERLEAVED)
                        acc = acc + qa * ka + qb * kb
                    return acc
                acc = lax.fori_loop(0, DH // 8, dbody, zero16)
                compute(lids, h, acc)

    base0 = ebase
    pltpu.sync_copy(src_hbm.at[pl.ds(base0, C)], srcv)
    pltpu.sync_copy(dst_hbm.at[pl.ds(base0, C)], dstv)
    pltpu.async_copy(q_hbm.at[dstv], qrows, sem).wait()
    pltpu.async_copy(kf_hbm.at[srcv], krows, sem).wait()
    alpha_chunk(lambda lids, h, acc:
                plsc.store_scatter(abuf, [lids * H + h], acc))
    def mbody(r, mm):
        return jnp.maximum(mm, abuf[pl.ds(r * 16, 16)])
    mfin = lax.fori_loop(0, C // 2, mbody, zero16)

    mvec = zero16
    for h in range(H):
        mh = jnp.maximum(mfin[h], mfin[h + 8])
        mvec = jnp.where(lane == h, mh, mvec)
    mbuf[...] = mvec
    pltpu.sync_copy(mbuf, msh.at[sid])
    plsc.subcore_barrier()
    pltpu.sync_copy(msh, mall)
    msc = mall[0, pl.ds(0, 16)]
    for j in range(1, NS):
        msc = jnp.maximum(msc, mall[j, pl.ds(0, 16)])
    mbuf[...] = msc
    mh_scalar = [msc[h] for h in range(H)]

    def process(lids, h, acc):
        hcol = jnp.full((16,), h, jnp.int32)
        ev = jnp.exp(acc - mh_scalar[h])
        plsc.store_scatter(ebuf, [lids, hcol], ev)
        def dbody(t, carry, _h=h, _lids=lids, _ev=ev):
            colp = jnp.broadcast_to(_h * (DH // 2) + t * 4,
                                    (16,)).astype(jnp.int32)
            colm = jnp.broadcast_to(_h * DH + t * 8, (16,)).astype(jnp.int32)
            for dd in range(4):
                vw = plsc.load_gather(vrows, [_lids, colp + dd])
                va, vb = plsc.unpack(plsc.bitcast(vw, jnp.bfloat16),
                                     format=plsc.PackFormat.INTERLEAVED)
                plsc.store_scatter(msgb, [_lids, colm + 2 * dd], va * _ev)
                plsc.store_scatter(msgb, [_lids, colm + 2 * dd + 1], vb * _ev)
            return carry
        lax.fori_loop(0, DH // 8, dbody, 0)

    def chunk_main(i, carry):
        base = ebase + i * C
        pltpu.sync_copy(src_hbm.at[pl.ds(base, C)], srcv)
        pltpu.sync_copy(dst_hbm.at[pl.ds(base, C)], dstv)
        pltpu.async_copy(q_hbm.at[dstv], qrows, sem).wait()
        pltpu.async_copy(kf_hbm.at[srcv], krows, sem).wait()
        pltpu.async_copy(vf_hbm.at[srcv], vrows, sem).wait()
        alpha_chunk(process)
        pltpu.sync_copy(msgb, acc_sh.at[dstv], add=True)
        pltpu.sync_copy(ebuf, s_sh.at[dstv], add=True)
        return carry

    lax.fori_loop(0, nch, chunk_main, 0)
    plsc.subcore_barrier()

    pltpu.sync_copy(acc_sh.at[pl.ds(row0, rows_per)],
                    acc_hbm.at[cid, pl.ds(row0, rows_per)])
    pltpu.sync_copy(s_sh.at[pl.ds(row0, rows_per)],
                    s_hbm.at[cid, pl.ds(row0, rows_per)])

    @pl.when(sid == 0)
    def _():
        pltpu.sync_copy(mbuf, m_hbm.at[cid])


def _edge_phase(q_pad, kf, vf, srcp, dstp, npad, ew, nch):
    mesh = plsc.VectorSubcoreMesh(core_axis_name="c", subcore_axis_name="s")
    body = functools.partial(_edge_body, npad, ew, nch)
    f = pl.kernel(
        body,
        compiler_params=pltpu.CompilerParams(
            needs_layout_passes=False, use_tc_tiling_on_sc=False),
        out_type=[
            jax.ShapeDtypeStruct((NC, npad, HID), jnp.float32),
            jax.ShapeDtypeStruct((NC, npad, 16), jnp.float32),
            jax.ShapeDtypeStruct((NC, 16), jnp.float32),
        ],
        mesh=mesh,
        scratch_types=[
            pltpu.VMEM((C,), jnp.int32),
            pltpu.VMEM((C,), jnp.int32),
            pltpu.VMEM((C, HID // 2), jnp.int32),
            pltpu.VMEM((C, HID // 2), jnp.int32),
            pltpu.VMEM((C, HID // 2), jnp.int32),
            pltpu.VMEM((C, HID), jnp.float32),
            pltpu.VMEM((C, 16), jnp.float32),
            pltpu.VMEM((C * H,), jnp.float32),
            pltpu.VMEM((16,), jnp.float32),
            pltpu.VMEM((NS, 16), jnp.float32),
            pltpu.VMEM_SHARED((npad, HID), jnp.float32),
            pltpu.VMEM_SHARED((npad, 16), jnp.float32),
            pltpu.VMEM_SHARED((NS, 16), jnp.float32),
            pltpu.SemaphoreType.DMA,
        ],
    )
    return f(q_pad, kf, vf, srcp, dstp)


def _finish_body(acc_ref, s_ref, hp_ref, sc0_ref, sc1_ref, ss0_ref, ss1_ref,
                 r_ref, rel_ref, wa_ref, ba_ref, w1_ref, w2_ref, bo_ref,
                 o_ref):
    a = acc_ref[0] * sc0_ref[...] + acc_ref[1] * sc1_ref[...]
    s = s_ref[0] * ss0_ref[...] + s_ref[1] * ss1_ref[...]
    den = s @ r_ref[...] + 1e-16
    outp = a / den + rel_ref[...]
    o = 0.5 * outp * (1.0 + lax.erf(outp / np.sqrt(2.0).astype(np.float32)))
    o2 = o @ wa_ref[...] + ba_ref[...]
    o_ref[...] = hp_ref[...] @ w1_ref[...] + o2 @ w2_ref[...] + bo_ref[...]


def _finish(acc, s, hp, sc, ss, rmat, rel, wa, ba, w1, w2, bo):
    n = hp.shape[0]
    dout = w1.shape[1]
    blk = _row_blocks(n)
    return pl.pallas_call(
        _finish_body,
        grid=(n // blk,),
        in_specs=[
            pl.BlockSpec((NC, blk, HID), lambda i: (0, i, 0)),
            pl.BlockSpec((NC, blk, 16), lambda i: (0, i, 0)),
            pl.BlockSpec((blk, HID), lambda i: (i, 0)),
            pl.BlockSpec((1, HID), lambda i: (0, 0)),
            pl.BlockSpec((1, HID), lambda i: (0, 0)),
            pl.BlockSpec((1, 16), lambda i: (0, 0)),
            pl.BlockSpec((1, 16), lambda i: (0, 0)),
            pl.BlockSpec((16, HID), lambda i: (0, 0)),
            pl.BlockSpec((1, HID), lambda i: (0, 0)),
            pl.BlockSpec((HID, HID), lambda i: (0, 0)),
            pl.BlockSpec((1, HID), lambda i: (0, 0)),
            pl.BlockSpec((HID, dout), lambda i: (0, 0)),
            pl.BlockSpec((HID, dout), lambda i: (0, 0)),
            pl.BlockSpec((1, dout), lambda i: (0, 0)),
        ],
        out_specs=pl.BlockSpec((blk, dout), lambda i: (i, 0)),
        out_shape=jax.ShapeDtypeStruct((n, dout), jnp.float32),
    )(acc, s, hp, sc[0:1], sc[1:2], ss[0:1], ss[1:2], rmat, rel, wa, ba,
      w1, w2, bo)


def _block_diag(a):
    bd = jnp.zeros((H, DH, H, DH), jnp.float32)
    bd = bd.at[jnp.arange(H), :, jnp.arange(H), :].set(a)
    return bd.reshape(HID, HID)


def kernel(x_paper, x_author, ei_writes, ei_written_by, params):
    n_p = x_paper.shape[0]
    e = ei_writes.shape[1]

    wi_p, bi_p = params['in_paper']
    wi_a, bi_a = params['in_author']
    wq, bq = params['q_paper']
    wk, bk = params['k_author']
    wv, bv = params['v_author']
    bda = _block_diag(params['a_rel_writes'])
    bdm = _block_diag(params['m_rel_writes'])
    qscale = jnp.repeat(params['p_rel_writes'], DH) / np.sqrt(DH).astype(np.float32)
    wq_f = wq * qscale[None, :]
    bq_f = (bq * qscale)[None, :]
    wk_f = wk @ bda
    bk_f = (bk @ bda)[None, :]
    wv_f = wv @ bdm
    bv_f = (bv @ bdm)[None, :]

    h_p, q = _proj_paper(x_paper, wi_p, bi_p[None, :], wq_f, bq_f)
    kf, vf = _proj_author(x_author, wi_a, bi_a[None, :], wk_f, bk_f, wv_f, bv_f)

    ew = -(-e // (NW * C)) * C
    e_pad = ew * NW
    npad = -(-(n_p + 1) // 128) * 128
    src = ei_writes[0]
    dst = ei_writes[1]
    if e_pad > e:
        src = jnp.concatenate([src, jnp.zeros((e_pad - e,), src.dtype)])
        dst = jnp.concatenate([dst, jnp.full((e_pad - e,), n_p, dst.dtype)])
    q_pad = jnp.concatenate([q, jnp.zeros((npad - n_p, HID), jnp.bfloat16)])
    q_i32 = lax.bitcast_convert_type(
        q_pad.reshape(npad, HID // 2, 2), jnp.int32)
    kf_i32 = lax.bitcast_convert_type(
        kf.reshape(-1, HID // 2, 2), jnp.int32)
    vf_i32 = lax.bitcast_convert_type(
        vf.reshape(-1, HID // 2, 2), jnp.int32)

    acc, s, m = _edge_phase(q_i32, kf_i32, vf_i32, src, dst, npad, ew,
                            ew // C)

    mmax = jnp.max(m, axis=0)
    ss = jnp.exp(m - mmax[None, :])
    sc = jnp.repeat(ss[:, :H], DH, axis=1)
    rmat = jnp.repeat(jnp.eye(16, dtype=jnp.float32)[:, :H], DH, axis=1)

    wa, ba = params['a_paper']
    wo, bo = params['out']
    out = _finish(acc, s, h_p,
                  sc, ss, rmat,
                  params['rel_enc_writes'][None, :],
                  wa, ba[None, :], wo[:HID], wo[HID:], bo[None, :])
    return out


# speedup vs baseline: 4.3686x; 1.1736x over previous
import functools

import jax
import jax.numpy as jnp
import numpy as np
from jax import lax
from jax.experimental import pallas as pl
from jax.experimental.pallas import tpu as pltpu
from jax.experimental.pallas import tpu_sc as plsc

H = 8
HID = 128
DH = HID // H

NC = 2
NS = 16
NW = NC * NS
C = 48


def _proj_paper_body(x_ref, wi_ref, bi_ref, wq_ref, bq_ref, h_ref, q_ref):
    h = jax.nn.relu(x_ref[...] @ wi_ref[...] + bi_ref[...])
    h_ref[...] = h
    q_ref[...] = (h @ wq_ref[...] + bq_ref[...]).astype(jnp.bfloat16)


def _proj_author_body(x_ref, wi_ref, bi_ref, wk_ref, bk_ref, wv_ref, bv_ref,
                      k_ref, v_ref):
    h = jax.nn.relu(x_ref[...] @ wi_ref[...] + bi_ref[...])
    k_ref[...] = (h @ wk_ref[...] + bk_ref[...]).astype(jnp.bfloat16)
    v_ref[...] = (h @ wv_ref[...] + bv_ref[...]).astype(jnp.bfloat16)


def _row_blocks(n):
    for b in (1000, 500, 250, 200, 125, 100, 50, 40, 25, 20, 10, 8, 5, 4, 2, 1):
        if n % b == 0:
            return b
    return 1


def _proj_paper(x, wi, bi, wq, bq):
    n, din = x.shape
    blk = _row_blocks(n)
    return pl.pallas_call(
        _proj_paper_body,
        grid=(n // blk,),
        in_specs=[
            pl.BlockSpec((blk, din), lambda i: (i, 0)),
            pl.BlockSpec((din, HID), lambda i: (0, 0)),
            pl.BlockSpec((1, HID), lambda i: (0, 0)),
            pl.BlockSpec((HID, HID), lambda i: (0, 0)),
            pl.BlockSpec((1, HID), lambda i: (0, 0)),
        ],
        out_specs=[
            pl.BlockSpec((blk, HID), lambda i: (i, 0)),
            pl.BlockSpec((blk, HID), lambda i: (i, 0)),
        ],
        out_shape=[
            jax.ShapeDtypeStruct((n, HID), jnp.float32),
            jax.ShapeDtypeStruct((n, HID), jnp.bfloat16),
        ],
    )(x, wi, bi, wq, bq)


def _proj_author(x, wi, bi, wk, bk, wv, bv):
    n, din = x.shape
    blk = _row_blocks(n)
    return pl.pallas_call(
        _proj_author_body,
        grid=(n // blk,),
        in_specs=[
            pl.BlockSpec((blk, din), lambda i: (i, 0)),
            pl.BlockSpec((din, HID), lambda i: (0, 0)),
            pl.BlockSpec((1, HID), lambda i: (0, 0)),
            pl.BlockSpec((HID, HID), lambda i: (0, 0)),
            pl.BlockSpec((1, HID), lambda i: (0, 0)),
            pl.BlockSpec((HID, HID), lambda i: (0, 0)),
            pl.BlockSpec((1, HID), lambda i: (0, 0)),
        ],
        out_specs=[
            pl.BlockSpec((blk, HID), lambda i: (i, 0)),
            pl.BlockSpec((blk, HID), lambda i: (i, 0)),
        ],
        out_shape=[
            jax.ShapeDtypeStruct((n, HID), jnp.bfloat16),
            jax.ShapeDtypeStruct((n, HID), jnp.bfloat16),
        ],
    )(x, wi, bi, wk, bk, wv, bv)


def _edge_body(npad, ew, nch,
               q_hbm, kf_hbm, vf_hbm, src_hbm, dst_hbm,
               acc_hbm, s_hbm, m_hbm,
               srcv, dstv, qrows, krows, vrows,
               srcv2, dstv2, qrows2, krows2, vrows2,
               msgb, ebuf, abuf, mbuf, mall,
               acc_sh, s_sh, msh,
               semq0, semk0, semv0, semq1, semk1, semv1):
    cid = lax.axis_index("c")
    sid = lax.axis_index("s")
    wid = cid * NS + sid
    ebase = wid * ew
    rows_per = npad // NS
    lane = lax.iota(jnp.int32, 16)
    zero16 = jnp.zeros((16,), jnp.float32)

    for r in range(C):
        ebuf[r, pl.ds(0, 16)] = zero16
    for r in range(C):
        for j in range(8):
            msgb[r, pl.ds(j * 16, 16)] = zero16
    row0 = sid * rows_per
    done = 0
    while done < rows_per:
        nr = min(C, rows_per - done)
        pltpu.sync_copy(msgb.at[pl.ds(0, nr)], acc_sh.at[pl.ds(row0 + done, nr)])
        pltpu.sync_copy(ebuf.at[pl.ds(0, nr)], s_sh.at[pl.ds(row0 + done, nr)])
        done += nr

    def alpha_chunk_buf(compute, qr, kr):
        for g in range(C // 16):
            lids = lane + (g * 16)
            for h in range(H):
                def dbody(t, acc, _h=h, _lids=lids):
                    col0 = jnp.broadcast_to(_h * (DH // 2) + t * 4,
                                            (16,)).astype(jnp.int32)
                    for dd in range(4):
                        qw = plsc.load_gather(qr, [_lids, col0 + dd])
                        kw = plsc.load_gather(kr, [_lids, col0 + dd])
                        qa, qb = plsc.unpack(plsc.bitcast(qw, jnp.bfloat16),
                                             format=plsc.PackFormat.INTERLEAVED)
                        ka, kb = plsc.unpack(plsc.bitcast(kw, jnp.bfloat16),
                                             format=plsc.PackFormat.INTERLEAVED)
                        acc = acc + qa * ka + qb * kb
                    return acc
                acc = lax.fori_loop(0, DH // 8, dbody, zero16)
                compute(lids, h, acc)

    base0 = ebase
    pltpu.sync_copy(src_hbm.at[pl.ds(base0, C)], srcv)
    pltpu.sync_copy(dst_hbm.at[pl.ds(base0, C)], dstv)
    pltpu.async_copy(q_hbm.at[dstv], qrows, semq0).wait()
    pltpu.async_copy(kf_hbm.at[srcv], krows, semk0).wait()
    alpha_chunk_buf(lambda lids, h, acc:
                    plsc.store_scatter(abuf, [lids * H + h], acc),
                    qrows, krows)
    def mbody(r, mm):
        return jnp.maximum(mm, abuf[pl.ds(r * 16, 16)])
    mfin = lax.fori_loop(0, C // 2, mbody, zero16)

    mvec = zero16
    for h in range(H):
        mh = jnp.maximum(mfin[h], mfin[h + 8])
        mvec = jnp.where(lane == h, mh, mvec)
    mbuf[...] = mvec
    pltpu.sync_copy(mbuf, msh.at[sid])
    plsc.subcore_barrier()
    pltpu.sync_copy(msh, mall)
    msc = mall[0, pl.ds(0, 16)]
    for j in range(1, NS):
        msc = jnp.maximum(msc, mall[j, pl.ds(0, 16)])
    mbuf[...] = msc
    mh_scalar = [msc[h] for h in range(H)]

    def process(lids, h, acc, vr):
        hcol = jnp.full((16,), h, jnp.int32)
        ev = jnp.exp(acc - mh_scalar[h])
        plsc.store_scatter(ebuf, [lids, hcol], ev)
        def dbody(t, carry, _h=h, _lids=lids, _ev=ev):
            colp = jnp.broadcast_to(_h * (DH // 2) + t * 4,
                                    (16,)).astype(jnp.int32)
            colm = jnp.broadcast_to(_h * DH + t * 8, (16,)).astype(jnp.int32)
            for dd in range(4):
                vw = plsc.load_gather(vr, [_lids, colp + dd])
                va, vb = plsc.unpack(plsc.bitcast(vw, jnp.bfloat16),
                                     format=plsc.PackFormat.INTERLEAVED)
                plsc.store_scatter(msgb, [_lids, colm + 2 * dd], va * _ev)
                plsc.store_scatter(msgb, [_lids, colm + 2 * dd + 1], vb * _ev)
            return carry
        lax.fori_loop(0, DH // 8, dbody, 0)

    srHow = (srcv, srcv2)
    dsts = (dstv, dstv2)
    qb = (qrows, qrows2)
    kb = (krows, krows2)
    vb = (vrows, vrows2)
    sq = (semq0, semq1)
    sk = (semk0, semk1)
    sv = (semv0, semv1)

    def start(i, p):
        base = ebase + i * C
        pltpu.sync_copy(src_hbm.at[pl.ds(base, C)], srHow[p])
        pltpu.sync_copy(dst_hbm.at[pl.ds(base, C)], dsts[p])
        pltpu.async_copy(q_hbm.at[dsts[p]], qb[p], sq[p])
        pltpu.async_copy(kf_hbm.at[srHow[p]], kb[p], sk[p])
        pltpu.async_copy(vf_hbm.at[srHow[p]], vb[p], sv[p])

    def finish(i, p):
        base = ebase + i * C
        pltpu.make_async_copy(q_hbm.at[dsts[p]], qb[p], sq[p]).wait()
        pltpu.make_async_copy(kf_hbm.at[srHow[p]], kb[p], sk[p]).wait()
        pltpu.make_async_copy(vf_hbm.at[srHow[p]], vb[p], sv[p]).wait()

    start(0, 0)

    def chunk_main(ii, carry):
        for p in range(2):
            i = 2 * ii + p
            start(i + 1, p ^ 1)
            finish(i, p)
            alpha_chunk_buf(
                lambda lids, h, acc, _vr=vb[p]: process(lids, h, acc, _vr),
                qb[p], kb[p])
            pltpu.sync_copy(msgb, acc_sh.at[dsts[p]], add=True)
            pltpu.sync_copy(ebuf, s_sh.at[dsts[p]], add=True)
        return carry

    lax.fori_loop(0, nch // 2, chunk_main, 0)
    finish(nch, 0)
    plsc.subcore_barrier()

    pltpu.sync_copy(acc_sh.at[pl.ds(row0, rows_per)],
                    acc_hbm.at[cid, pl.ds(row0, rows_per)])
    pltpu.sync_copy(s_sh.at[pl.ds(row0, rows_per)],
                    s_hbm.at[cid, pl.ds(row0, rows_per)])

    @pl.when(sid == 0)
    def _():
        pltpu.sync_copy(mbuf, m_hbm.at[cid])


def _edge_phase(q_pad, kf, vf, srcp, dstp, npad, ew, nch):
    mesh = plsc.VectorSubcoreMesh(core_axis_name="c", subcore_axis_name="s")
    body = functools.partial(_edge_body, npad, ew, nch)
    f = pl.kernel(
        body,
        compiler_params=pltpu.CompilerParams(
            needs_layout_passes=False, use_tc_tiling_on_sc=False),
        out_type=[
            jax.ShapeDtypeStruct((NC, npad, HID), jnp.float32),
            jax.ShapeDtypeStruct((NC, npad, 16), jnp.float32),
            jax.ShapeDtypeStruct((NC, 16), jnp.float32),
        ],
        mesh=mesh,
        scratch_types=[
            pltpu.VMEM((C,), jnp.int32),
            pltpu.VMEM((C,), jnp.int32),
            pltpu.VMEM((C, HID // 2), jnp.int32),
            pltpu.VMEM((C, HID // 2), jnp.int32),
            pltpu.VMEM((C, HID // 2), jnp.int32),
            pltpu.VMEM((C,), jnp.int32),
            pltpu.VMEM((C,), jnp.int32),
            pltpu.VMEM((C, HID // 2), jnp.int32),
            pltpu.VMEM((C, HID // 2), jnp.int32),
            pltpu.VMEM((C, HID // 2), jnp.int32),
            pltpu.VMEM((C, HID), jnp.float32),
            pltpu.VMEM((C, 16), jnp.float32),
            pltpu.VMEM((C * H,), jnp.float32),
            pltpu.VMEM((16,), jnp.float32),
            pltpu.VMEM((NS, 16), jnp.float32),
            pltpu.VMEM_SHARED((npad, HID), jnp.float32),
            pltpu.VMEM_SHARED((npad, 16), jnp.float32),
            pltpu.VMEM_SHARED((NS, 16), jnp.float32),
            pltpu.SemaphoreType.DMA,
            pltpu.SemaphoreType.DMA,
            pltpu.SemaphoreType.DMA,
            pltpu.SemaphoreType.DMA,
            pltpu.SemaphoreType.DMA,
            pltpu.SemaphoreType.DMA,
        ],
    )
    return f(q_pad, kf, vf, srcp, dstp)


def _finish_body(acc_ref, s_ref, hp_ref, sc0_ref, sc1_ref, ss0_ref, ss1_ref,
                 r_ref, rel_ref, wa_ref, ba_ref, w1_ref, w2_ref, bo_ref,
                 o_ref):
    a = acc_ref[0] * sc0_ref[...] + acc_ref[1] * sc1_ref[...]
    s = s_ref[0] * ss0_ref[...] + s_ref[1] * ss1_ref[...]
    den = s @ r_ref[...] + 1e-16
    outp = a / den + rel_ref[...]
    o = 0.5 * outp * (1.0 + lax.erf(outp / np.sqrt(2.0).astype(np.float32)))
    o2 = o @ wa_ref[...] + ba_ref[...]
    o_ref[...] = hp_ref[...] @ w1_ref[...] + o2 @ w2_ref[...] + bo_ref[...]


def _finish(acc, s, hp, sc, ss, rmat, rel, wa, ba, w1, w2, bo):
    n = hp.shape[0]
    dout = w1.shape[1]
    blk = _row_blocks(n)
    return pl.pallas_call(
        _finish_body,
        grid=(n // blk,),
        in_specs=[
            pl.BlockSpec((NC, blk, HID), lambda i: (0, i, 0)),
            pl.BlockSpec((NC, blk, 16), lambda i: (0, i, 0)),
            pl.BlockSpec((blk, HID), lambda i: (i, 0)),
            pl.BlockSpec((1, HID), lambda i: (0, 0)),
            pl.BlockSpec((1, HID), lambda i: (0, 0)),
            pl.BlockSpec((1, 16), lambda i: (0, 0)),
            pl.BlockSpec((1, 16), lambda i: (0, 0)),
            pl.BlockSpec((16, HID), lambda i: (0, 0)),
            pl.BlockSpec((1, HID), lambda i: (0, 0)),
            pl.BlockSpec((HID, HID), lambda i: (0, 0)),
            pl.BlockSpec((1, HID), lambda i: (0, 0)),
            pl.BlockSpec((HID, dout), lambda i: (0, 0)),
            pl.BlockSpec((HID, dout), lambda i: (0, 0)),
            pl.BlockSpec((1, dout), lambda i: (0, 0)),
        ],
        out_specs=pl.BlockSpec((blk, dout), lambda i: (i, 0)),
        out_shape=jax.ShapeDtypeStruct((n, dout), jnp.float32),
    )(acc, s, hp, sc[0:1], sc[1:2], ss[0:1], ss[1:2], rmat, rel, wa, ba,
      w1, w2, bo)


def _block_diag(a):
    bd = jnp.zeros((H, DH, H, DH), jnp.float32)
    bd = bd.at[jnp.arange(H), :, jnp.arange(H), :].set(a)
    return bd.reshape(HID, HID)


def kernel(x_paper, x_author, ei_writes, ei_written_by, params):
    n_p = x_paper.shape[0]
    e = ei_writes.shape[1]

    wi_p, bi_p = params['in_paper']
    wi_a, bi_a = params['in_author']
    wq, bq = params['q_paper']
    wk, bk = params['k_author']
    wv, bv = params['v_author']
    bda = _block_diag(params['a_rel_writes'])
    bdm = _block_diag(params['m_rel_writes'])
    qscale = jnp.repeat(params['p_rel_writes'], DH) / np.sqrt(DH).astype(np.float32)
    wq_f = wq * qscale[None, :]
    bq_f = (bq * qscale)[None, :]
    wk_f = wk @ bda
    bk_f = (bk @ bda)[None, :]
    wv_f = wv @ bdm
    bv_f = (bv @ bdm)[None, :]

    h_p, q = _proj_paper(x_paper, wi_p, bi_p[None, :], wq_f, bq_f)
    kf, vf = _proj_author(x_author, wi_a, bi_a[None, :], wk_f, bk_f, wv_f, bv_f)

    ew = -(-e // (NW * 2 * C)) * (2 * C)
    e_pad = ew * NW + C
    npad = -(-(n_p + 1) // 128) * 128
    src = ei_writes[0]
    dst = ei_writes[1]
    if e_pad > e:
        src = jnp.concatenate([src, jnp.zeros((e_pad - e,), src.dtype)])
        dst = jnp.concatenate([dst, jnp.full((e_pad - e,), n_p, dst.dtype)])
    q_pad = jnp.concatenate([q, jnp.zeros((npad - n_p, HID), jnp.bfloat16)])
    q_i32 = lax.bitcast_convert_type(
        q_pad.reshape(npad, HID // 2, 2), jnp.int32)
    kf_i32 = lax.bitcast_convert_type(
        kf.reshape(-1, HID // 2, 2), jnp.int32)
    vf_i32 = lax.bitcast_convert_type(
        vf.reshape(-1, HID // 2, 2), jnp.int32)

    acc, s, m = _edge_phase(q_i32, kf_i32, vf_i32, src, dst, npad, ew,
                            ew // C)

    mmax = jnp.max(m, axis=0)
    ss = jnp.exp(m - mmax[None, :])
    sc = jnp.repeat(ss[:, :H], DH, axis=1)
    rmat = jnp.repeat(jnp.eye(16, dtype=jnp.float32)[:, :H], DH, axis=1)

    wa, ba = params['a_paper']
    wo, bo = params['out']
    out = _finish(acc, s, h_p,
                  sc, ss, rmat,
                  params['rel_enc_writes'][None, :],
                  wa, ba[None, :], wo[:HID], wo[HID:], bo[None, :])
    return out


# speedup vs baseline: 5.2797x; 1.2086x over previous
import functools

import jax
import jax.numpy as jnp
import numpy as np
from jax import lax
from jax.experimental import pallas as pl
from jax.experimental.pallas import tpu as pltpu
from jax.experimental.pallas import tpu_sc as plsc

H = 8
HID = 128
DH = HID // H

NC = 2
NS = 16
NW = NC * NS
C = 40


def _proj_paper_body(x_ref, wi_ref, bi_ref, wq_ref, bq_ref, h_ref, q_ref):
    h = jax.nn.relu(x_ref[...] @ wi_ref[...] + bi_ref[...])
    h_ref[...] = h
    q_ref[...] = (h @ wq_ref[...] + bq_ref[...]).astype(jnp.bfloat16)


def _proj_author_body(x_ref, wi_ref, bi_ref, wk_ref, bk_ref, wv_ref, bv_ref,
                      k_ref, v_ref):
    h = jax.nn.relu(x_ref[...] @ wi_ref[...] + bi_ref[...])
    k_ref[...] = (h @ wk_ref[...] + bk_ref[...]).astype(jnp.bfloat16)
    v_ref[...] = (h @ wv_ref[...] + bv_ref[...]).astype(jnp.bfloat16)


def _row_blocks(n):
    for b in (1000, 500, 250, 200, 125, 100, 50, 40, 25, 20, 10, 8, 5, 4, 2, 1):
        if n % b == 0:
            return b
    return 1


def _proj_paper(x, wi, bi, wq, bq):
    n, din = x.shape
    blk = _row_blocks(n)
    return pl.pallas_call(
        _proj_paper_body,
        grid=(n // blk,),
        in_specs=[
            pl.BlockSpec((blk, din), lambda i: (i, 0)),
            pl.BlockSpec((din, HID), lambda i: (0, 0)),
            pl.BlockSpec((1, HID), lambda i: (0, 0)),
            pl.BlockSpec((HID, HID), lambda i: (0, 0)),
            pl.BlockSpec((1, HID), lambda i: (0, 0)),
        ],
        out_specs=[
            pl.BlockSpec((blk, HID), lambda i: (i, 0)),
            pl.BlockSpec((blk, HID), lambda i: (i, 0)),
        ],
        out_shape=[
            jax.ShapeDtypeStruct((n, HID), jnp.float32),
            jax.ShapeDtypeStruct((n, HID), jnp.bfloat16),
        ],
    )(x, wi, bi, wq, bq)


def _proj_author(x, wi, bi, wk, bk, wv, bv):
    n, din = x.shape
    blk = _row_blocks(n)
    return pl.pallas_call(
        _proj_author_body,
        grid=(n // blk,),
        in_specs=[
            pl.BlockSpec((blk, din), lambda i: (i, 0)),
            pl.BlockSpec((din, HID), lambda i: (0, 0)),
            pl.BlockSpec((1, HID), lambda i: (0, 0)),
            pl.BlockSpec((HID, HID), lambda i: (0, 0)),
            pl.BlockSpec((1, HID), lambda i: (0, 0)),
            pl.BlockSpec((HID, HID), lambda i: (0, 0)),
            pl.BlockSpec((1, HID), lambda i: (0, 0)),
        ],
        out_specs=[
            pl.BlockSpec((blk, HID), lambda i: (i, 0)),
            pl.BlockSpec((blk, HID), lambda i: (i, 0)),
        ],
        out_shape=[
            jax.ShapeDtypeStruct((n, HID), jnp.bfloat16),
            jax.ShapeDtypeStruct((n, HID), jnp.bfloat16),
        ],
    )(x, wi, bi, wk, bk, wv, bv)


def _edge_body(npad, ew, nch,
               q_hbm, kf_hbm, vf_hbm, src_hbm, dst_hbm,
               acc_hbm, s_hbm, m_hbm,
               srcv, dstv, qrows, krows, vrows,
               srcv2, dstv2, qrows2, krows2, vrows2,
               msgb, msgb2, ebuf, ebuf2, abuf, mbuf, mall,
               acc_sh, s_sh, msh,
               semq0, semk0, semv0, semq1, semk1, semv1, semm0, semm1):
    cid = lax.axis_index("c")
    sid = lax.axis_index("s")
    wid = cid * NS + sid
    ebase = wid * ew
    rows_per = npad // NS
    lane = lax.iota(jnp.int32, 16)
    zero16 = jnp.zeros((16,), jnp.float32)

    for r in range(C):
        ebuf[r, pl.ds(0, 16)] = zero16
        ebuf2[r, pl.ds(0, 16)] = zero16
    for r in range(C):
        for j in range(8):
            msgb[r, pl.ds(j * 16, 16)] = zero16
    row0 = sid * rows_per
    done = 0
    while done < rows_per:
        nr = min(C, rows_per - done)
        pltpu.sync_copy(msgb.at[pl.ds(0, nr)], acc_sh.at[pl.ds(row0 + done, nr)])
        pltpu.sync_copy(ebuf.at[pl.ds(0, nr)], s_sh.at[pl.ds(row0 + done, nr)])
        done += nr

    def alpha_chunk_buf(compute, qr, kr):
        for g in range(C // 16):
            lids = lane + (g * 16)
            for h in range(H):
                def dbody(t, acc, _h=h, _lids=lids):
                    col0 = jnp.broadcast_to(_h * (DH // 2) + t * 4,
                                            (16,)).astype(jnp.int32)
                    for dd in range(4):
                        qw = plsc.load_gather(qr, [_lids, col0 + dd])
                        kw = plsc.load_gather(kr, [_lids, col0 + dd])
                        qa, qb = plsc.unpack(plsc.bitcast(qw, jnp.bfloat16),
                                             format=plsc.PackFormat.INTERLEAVED)
                        ka, kb = plsc.unpack(plsc.bitcast(kw, jnp.bfloat16),
                                             format=plsc.PackFormat.INTERLEAVED)
                        acc = acc + qa * ka + qb * kb
                    return acc
                acc = lax.fori_loop(0, DH // 8, dbody, zero16)
                compute(lids, h, acc)

    base0 = ebase
    pltpu.sync_copy(src_hbm.at[pl.ds(base0, C)], srcv)
    pltpu.sync_copy(dst_hbm.at[pl.ds(base0, C)], dstv)
    pltpu.async_copy(q_hbm.at[dstv], qrows, semq0).wait()
    pltpu.async_copy(kf_hbm.at[srcv], krows, semk0).wait()
    alpha_chunk_buf(lambda lids, h, acc:
                    plsc.store_scatter(abuf, [lids * H + h], acc),
                    qrows, krows)
    def mbody(r, mm):
        return jnp.maximum(mm, abuf[pl.ds(r * 16, 16)])
    mfin = lax.fori_loop(0, C // 2, mbody, zero16)

    mvec = zero16
    for h in range(H):
        mh = jnp.maximum(mfin[h], mfin[h + 8])
        mvec = jnp.where(lane == h, mh, mvec)
    mbuf[...] = mvec
    pltpu.sync_copy(mbuf, msh.at[sid])
    plsc.subcore_barrier()
    pltpu.sync_copy(msh, mall)
    msc = mall[0, pl.ds(0, 16)]
    for j in range(1, NS):
        msc = jnp.maximum(msc, mall[j, pl.ds(0, 16)])
    mbuf[...] = msc
    mh_scalar = [msc[h] for h in range(H)]

    def process(lids, h, acc, vr, mr, er):
        hcol = jnp.full((16,), h, jnp.int32)
        ev = jnp.exp(acc - mh_scalar[h])
        plsc.store_scatter(er, [lids, hcol], ev)
        def dbody(t, carry, _h=h, _lids=lids, _ev=ev):
            colp = jnp.broadcast_to(_h * (DH // 2) + t * 4,
                                    (16,)).astype(jnp.int32)
            colm = jnp.broadcast_to(_h * DH + t * 8, (16,)).astype(jnp.int32)
            for dd in range(4):
                vw = plsc.load_gather(vr, [_lids, colp + dd])
                va, vb = plsc.unpack(plsc.bitcast(vw, jnp.bfloat16),
                                     format=plsc.PackFormat.INTERLEAVED)
                plsc.store_scatter(mr, [_lids, colm + 2 * dd], va * _ev)
                plsc.store_scatter(mr, [_lids, colm + 2 * dd + 1], vb * _ev)
            return carry
        lax.fori_loop(0, DH // 8, dbody, 0)

    srHow = (srcv, srcv2)
    dsts = (dstv, dstv2)
    qb = (qrows, qrows2)
    kb = (krows, krows2)
    vb = (vrows, vrows2)
    sq = (semq0, semq1)
    sk = (semk0, semk1)
    sv = (semv0, semv1)

    def start(i, p):
        base = ebase + i * C
        pltpu.sync_copy(src_hbm.at[pl.ds(base, C)], srHow[p])
        pltpu.sync_copy(dst_hbm.at[pl.ds(base, C)], dsts[p])
        pltpu.async_copy(q_hbm.at[dsts[p]], qb[p], sq[p])
        pltpu.async_copy(kf_hbm.at[srHow[p]], kb[p], sk[p])
        pltpu.async_copy(vf_hbm.at[srHow[p]], vb[p], sv[p])

    def finish(i, p):
        base = ebase + i * C
        pltpu.make_async_copy(q_hbm.at[dsts[p]], qb[p], sq[p]).wait()
        pltpu.make_async_copy(kf_hbm.at[srHow[p]], kb[p], sk[p]).wait()
        pltpu.make_async_copy(vf_hbm.at[srHow[p]], vb[p], sv[p]).wait()

    start(0, 0)

    mb = (msgb, msgb2)
    eb = (ebuf, ebuf2)
    sm = (semm0, semm1)

    def chunk_main(ii, carry):
        for p in range(2):
            i = 2 * ii + p
            start(i + 1, p ^ 1)
            @pl.when(ii >= 1)
            def _(p=p):
                pltpu.make_async_copy(mb[p], acc_sh.at[dsts[p]], sm[p]).wait()
                pltpu.make_async_copy(eb[p], s_sh.at[dsts[p]], sm[p]).wait()
            finish(i, p)
            alpha_chunk_buf(
                lambda lids, h, acc, _vr=vb[p], _p=p:
                    process(lids, h, acc, _vr, mb[_p], eb[_p]),
                qb[p], kb[p])
            pltpu.async_copy(mb[p], acc_sh.at[dsts[p]], sm[p], add=True)
            pltpu.async_copy(eb[p], s_sh.at[dsts[p]], sm[p], add=True)
        return carry

    lax.fori_loop(0, nch // 2, chunk_main, 0)
    finish(nch, 0)
    for p in range(2):
        pltpu.make_async_copy(mb[p], acc_sh.at[dsts[p]], sm[p]).wait()
        pltpu.make_async_copy(eb[p], s_sh.at[dsts[p]], sm[p]).wait()
    plsc.subcore_barrier()

    pltpu.sync_copy(acc_sh.at[pl.ds(row0, rows_per)],
                    acc_hbm.at[cid, pl.ds(row0, rows_per)])
    pltpu.sync_copy(s_sh.at[pl.ds(row0, rows_per)],
                    s_hbm.at[cid, pl.ds(row0, rows_per)])

    @pl.when(sid == 0)
    def _():
        pltpu.sync_copy(mbuf, m_hbm.at[cid])


def _edge_phase(q_pad, kf, vf, srcp, dstp, npad, ew, nch):
    mesh = plsc.VectorSubcoreMesh(core_axis_name="c", subcore_axis_name="s")
    body = functools.partial(_edge_body, npad, ew, nch)
    f = pl.kernel(
        body,
        compiler_params=pltpu.CompilerParams(
            needs_layout_passes=False, use_tc_tiling_on_sc=False),
        out_type=[
            jax.ShapeDtypeStruct((NC, npad, HID), jnp.float32),
            jax.ShapeDtypeStruct((NC, npad, 16), jnp.float32),
            jax.ShapeDtypeStruct((NC, 16), jnp.float32),
        ],
        mesh=mesh,
        scratch_types=[
            pltpu.VMEM((C,), jnp.int32),
            pltpu.VMEM((C,), jnp.int32),
            pltpu.VMEM((C, HID // 2), jnp.int32),
            pltpu.VMEM((C, HID // 2), jnp.int32),
            pltpu.VMEM((C, HID // 2), jnp.int32),
            pltpu.VMEM((C,), jnp.int32),
            pltpu.VMEM((C,), jnp.int32),
            pltpu.VMEM((C, HID // 2), jnp.int32),
            pltpu.VMEM((C, HID // 2), jnp.int32),
            pltpu.VMEM((C, HID // 2), jnp.int32),
            pltpu.VMEM((C, HID), jnp.float32),
            pltpu.VMEM((C, HID), jnp.float32),
            pltpu.VMEM((C, 16), jnp.float32),
            pltpu.VMEM((C, 16), jnp.float32),
            pltpu.VMEM((C * H,), jnp.float32),
            pltpu.VMEM((16,), jnp.float32),
            pltpu.VMEM((NS, 16), jnp.float32),
            pltpu.VMEM_SHARED((npad, HID), jnp.float32),
            pltpu.VMEM_SHARED((npad, 16), jnp.float32),
            pltpu.VMEM_SHARED((NS, 16), jnp.float32),
            pltpu.SemaphoreType.DMA,
            pltpu.SemaphoreType.DMA,
            pltpu.SemaphoreType.DMA,
            pltpu.SemaphoreType.DMA,
            pltpu.SemaphoreType.DMA,
            pltpu.SemaphoreType.DMA,
            pltpu.SemaphoreType.DMA,
            pltpu.SemaphoreType.DMA,
        ],
    )
    return f(q_pad, kf, vf, srcp, dstp)


def _finish_body(acc_ref, s_ref, hp_ref, sc0_ref, sc1_ref, ss0_ref, ss1_ref,
                 r_ref, rel_ref, wa_ref, ba_ref, w1_ref, w2_ref, bo_ref,
                 o_ref):
    a = acc_ref[0] * sc0_ref[...] + acc_ref[1] * sc1_ref[...]
    s = s_ref[0] * ss0_ref[...] + s_ref[1] * ss1_ref[...]
    den = s @ r_ref[...] + 1e-16
    outp = a / den + rel_ref[...]
    o = 0.5 * outp * (1.0 + lax.erf(outp / np.sqrt(2.0).astype(np.float32)))
    o2 = o @ wa_ref[...] + ba_ref[...]
    o_ref[...] = hp_ref[...] @ w1_ref[...] + o2 @ w2_ref[...] + bo_ref[...]


def _finish(acc, s, hp, sc, ss, rmat, rel, wa, ba, w1, w2, bo):
    n = hp.shape[0]
    dout = w1.shape[1]
    blk = _row_blocks(n)
    return pl.pallas_call(
        _finish_body,
        grid=(n // blk,),
        in_specs=[
            pl.BlockSpec((NC, blk, HID), lambda i: (0, i, 0)),
            pl.BlockSpec((NC, blk, 16), lambda i: (0, i, 0)),
            pl.BlockSpec((blk, HID), lambda i: (i, 0)),
            pl.BlockSpec((1, HID), lambda i: (0, 0)),
            pl.BlockSpec((1, HID), lambda i: (0, 0)),
            pl.BlockSpec((1, 16), lambda i: (0, 0)),
            pl.BlockSpec((1, 16), lambda i: (0, 0)),
            pl.BlockSpec((16, HID), lambda i: (0, 0)),
            pl.BlockSpec((1, HID), lambda i: (0, 0)),
            pl.BlockSpec((HID, HID), lambda i: (0, 0)),
            pl.BlockSpec((1, HID), lambda i: (0, 0)),
            pl.BlockSpec((HID, dout), lambda i: (0, 0)),
            pl.BlockSpec((HID, dout), lambda i: (0, 0)),
            pl.BlockSpec((1, dout), lambda i: (0, 0)),
        ],
        out_specs=pl.BlockSpec((blk, dout), lambda i: (i, 0)),
        out_shape=jax.ShapeDtypeStruct((n, dout), jnp.float32),
    )(acc, s, hp, sc[0:1], sc[1:2], ss[0:1], ss[1:2], rmat, rel, wa, ba,
      w1, w2, bo)


def _block_diag(a):
    bd = jnp.zeros((H, DH, H, DH), jnp.float32)
    bd = bd.at[jnp.arange(H), :, jnp.arange(H), :].set(a)
    return bd.reshape(HID, HID)


def kernel(x_paper, x_author, ei_writes, ei_written_by, params):
    n_p = x_paper.shape[0]
    e = ei_writes.shape[1]

    wi_p, bi_p = params['in_paper']
    wi_a, bi_a = params['in_author']
    wq, bq = params['q_paper']
    wk, bk = params['k_author']
    wv, bv = params['v_author']
    bda = _block_diag(params['a_rel_writes'])
    bdm = _block_diag(params['m_rel_writes'])
    qscale = jnp.repeat(params['p_rel_writes'], DH) / np.sqrt(DH).astype(np.float32)
    wq_f = wq * qscale[None, :]
    bq_f = (bq * qscale)[None, :]
    wk_f = wk @ bda
    bk_f = (bk @ bda)[None, :]
    wv_f = wv @ bdm
    bv_f = (bv @ bdm)[None, :]

    h_p, q = _proj_paper(x_paper, wi_p, bi_p[None, :], wq_f, bq_f)
    kf, vf = _proj_author(x_author, wi_a, bi_a[None, :], wk_f, bk_f, wv_f, bv_f)

    ew = -(-e // (NW * 2 * C)) * (2 * C)
    e_pad = ew * NW + C
    npad = -(-(n_p + 1) // 128) * 128
    src = ei_writes[0]
    dst = ei_writes[1]
    if e_pad > e:
        src = jnp.concatenate([src, jnp.zeros((e_pad - e,), src.dtype)])
        dst = jnp.concatenate([dst, jnp.full((e_pad - e,), n_p, dst.dtype)])
    q_pad = jnp.concatenate([q, jnp.zeros((npad - n_p, HID), jnp.bfloat16)])
    q_i32 = lax.bitcast_convert_type(
        q_pad.reshape(npad, HID // 2, 2), jnp.int32)
    kf_i32 = lax.bitcast_convert_type(
        kf.reshape(-1, HID // 2, 2), jnp.int32)
    vf_i32 = lax.bitcast_convert_type(
        vf.reshape(-1, HID // 2, 2), jnp.int32)

    acc, s, m = _edge_phase(q_i32, kf_i32, vf_i32, src, dst, npad, ew,
                            ew // C)

    mmax = jnp.max(m, axis=0)
    ss = jnp.exp(m - mmax[None, :])
    sc = jnp.repeat(ss[:, :H], DH, axis=1)
    rmat = jnp.repeat(jnp.eye(16, dtype=jnp.float32)[:, :H], DH, axis=1)

    wa, ba = params['a_paper']
    wo, bo = params['out']
    out = _finish(acc, s, h_p,
                  sc, ss, rmat,
                  params['rel_enc_writes'][None, :],
                  wa, ba[None, :], wo[:HID], wo[HID:], bo[None, :])
    return out
